# baseline (speedup 1.0000x reference)
import functools

import jax
import jax.numpy as jnp
from jax import lax
from jax.experimental import pallas as pl
from jax.experimental.pallas import tpu as pltpu
from jax.experimental.pallas import tpu_sc as plsc

_VOCAB = 1000
_B, _L = 1024, 50
_D = _VOCAB
_N = _B * _L
_NC, _NS = 2, 16
_NW = _NC * _NS
_NPW = _N // _NW
_C = 40
_NCHUNK = _NPW // _C


def _gather_body(idx_hbm, table_hbm, out_hbm, idx_v, rows_v, sem):
    wid = lax.axis_index("s") * _NC + lax.axis_index("c")
    base = wid * _NPW
    pltpu.sync_copy(idx_hbm.at[wid], idx_v)

    def chunk(g, carry):
        pltpu.async_copy(table_hbm.at[idx_v.at[g]], rows_v, sem).wait()
        pltpu.sync_copy(rows_v, out_hbm.at[pl.ds(base + g * _C, _C)])
        return carry

    lax.fori_loop(0, _NCHUNK, chunk, 0)


_gather = functools.partial(
    pl.kernel,
    mesh=plsc.VectorSubcoreMesh(core_axis_name="c", subcore_axis_name="s"),
    out_type=jax.ShapeDtypeStruct((_N, _D), jnp.float32),
    scratch_types=[
        pltpu.VMEM((_NCHUNK, _C), jnp.int32),
        pltpu.VMEM((_C, _D), jnp.float32),
        pltpu.SemaphoreType.DMA,
    ],
    compiler_params=pltpu.CompilerParams(use_tc_tiling_on_sc=False),
)(_gather_body)


def kernel(idx, logits_table):
    idx3 = idx.reshape(_NW, _NCHUNK, _C).astype(jnp.int32)
    out = _gather(idx3, logits_table)
    return out.reshape(_B, _L, _D)


# speedup vs baseline: 1.0382x; 1.0382x over previous
import functools

import jax
import jax.numpy as jnp
from jax import lax
from jax.experimental import pallas as pl
from jax.experimental.pallas import tpu as pltpu
from jax.experimental.pallas import tpu_sc as plsc

_VOCAB = 1000
_B, _L = 1024, 50
_D = _VOCAB
_N = _B * _L
_NC, _NS = 2, 16
_NW = _NC * _NS
_NPW = _N // _NW
_C = 40
_NCHUNK = _NPW // _C
_NBUF = 2


def _gather_body(idx_hbm, table_hbm, out_hbm, idx_v, *scratch):
    bufs = scratch[:_NBUF]
    sem_g = scratch[_NBUF:2 * _NBUF]
    sem_s = scratch[2 * _NBUF:3 * _NBUF]
    wid = lax.axis_index("s") * _NC + lax.axis_index("c")
    base = wid * _NPW
    pltpu.sync_copy(idx_hbm.at[wid], idx_v)

    def start_gather(c, b):
        pltpu.async_copy(table_hbm.at[idx_v.at[c]], bufs[b], sem_g[b])

    def wait_gather(c, b):
        pltpu.make_async_copy(table_hbm.at[idx_v.at[c]], bufs[b], sem_g[b]).wait()

    def start_store(c, b):
        pltpu.async_copy(bufs[b], out_hbm.at[pl.ds(base + c * _C, _C)], sem_s[b])

    def wait_store(c, b):
        pltpu.make_async_copy(
            bufs[b], out_hbm.at[pl.ds(base + c * _C, _C)], sem_s[b]
        ).wait()

    for b in range(_NBUF):
        start_gather(b, b)

    def outer(i, carry):
        for b in range(_NBUF):
            c = i * _NBUF + b
            wait_gather(c, b)
            start_store(c, b)

            @pl.when(c + _NBUF < _NCHUNK)
            def _():
                wait_store(c, b)
                start_gather(c + _NBUF, b)

        return carry

    lax.fori_loop(0, _NCHUNK // _NBUF, outer, 0)
    for b in range(_NBUF):
        wait_store(_NCHUNK - _NBUF + b, b)


_gather = functools.partial(
    pl.kernel,
    mesh=plsc.VectorSubcoreMesh(core_axis_name="c", subcore_axis_name="s"),
    out_type=jax.ShapeDtypeStruct((_N, _D), jnp.float32),
    scratch_types=[pltpu.VMEM((_NCHUNK, _C), jnp.int32)]
    + [pltpu.VMEM((_C, _D), jnp.float32) for _ in range(_NBUF)]
    + [pltpu.SemaphoreType.DMA for _ in range(2 * _NBUF)],
    compiler_params=pltpu.CompilerParams(use_tc_tiling_on_sc=False),
)(_gather_body)


def kernel(idx, logits_table):
    idx3 = idx.reshape(_NW, _NCHUNK, _C).astype(jnp.int32)
    out = _gather(idx3, logits_table)
    return out.reshape(_B, _L, _D)
